# baseline (speedup 1.0000x reference)
import functools

import jax
import jax.numpy as jnp
from jax import lax
from jax.experimental import pallas as pl
from jax.experimental.pallas import tpu as pltpu
from jax.experimental.pallas import tpu_sc as plsc

F32 = jnp.float32
BF16 = jnp.bfloat16
D = 128
NMESH = 10000
NHYPER = 1000
NNODE = NMESH + NHYPER
NC, NS = 2, 16
NW = NC * NS
CH = 128
EPS = 1e-5

_SC_MESH = dict(core_axis_name="c", subcore_axis_name="s",
                num_cores=NC, num_subcores=NS)


def _chunk_plan(per_worker):
    best = None
    for c in range(8, CH + 1, 8):
        if per_worker % c:
            continue
        n_ch = per_worker // c
        cap = min(8, 896 // (2 * c))
        gs = [g for g in range(1, cap + 1) if n_ch % g == 0]
        if not gs:
            continue
        g = max(gs)
        if best is None or (g, c) > (best[1], best[0]):
            best = (c, g, n_ch // g)
    return best


GCH = 40
GGRP = 5


@functools.lru_cache(maxsize=None)
def _make_gather_all(jobs, gch=GCH):
    plans = []
    for E, TR in jobs:
        per_w = E // NW
        assert per_w % gch == 0
        n_ch = per_w // gch
        grp = max(g for g in range(1, GGRP + 1) if n_ch % g == 0)
        assert TR % NS == 0
        plans.append((E, TR, per_w, grp, n_ch // grp, TR // NS))
    tr_max = max(TR for _, TR in jobs)
    mesh = plsc.VectorSubcoreMesh(**_SC_MESH)
    scratch = ([pltpu.VMEM((gch,), jnp.int32) for _ in range(GGRP)]
               + [pltpu.VMEM((gch, D), F32) for _ in range(GGRP)]
               + [pltpu.VMEM_SHARED((tr_max, D), F32),
                  pltpu.SemaphoreType.DMA, pltpu.SemaphoreType.DMA,
                  pltpu.SemaphoreType.DMA])
    out_type = tuple(jax.ShapeDtypeStruct((E, D), F32) for E, _ in jobs)

    @functools.partial(pl.kernel, out_type=out_type, mesh=mesh,
                       scratch_types=scratch)
    def gather_k(*refs):
        nj = len(jobs)
        args = refs[:2 * nj]
        outs = refs[2 * nj:3 * nj]
        scr = refs[3 * nj:]
        idxs = scr[:GGRP]
        rows = scr[GGRP:2 * GGRP]
        stage, sem_t, sem_i, sem_g = scr[2 * GGRP:]
        cid = lax.axis_index("c")
        sid = lax.axis_index("s")
        wid = sid * NC + cid

        for ji, (E, TR, per_w, grp, n_grp, tper) in enumerate(plans):
            tab_hbm, idx_hbm = args[2 * ji], args[2 * ji + 1]
            out = outs[ji]
            gw = gch * grp
            pltpu.async_copy(tab_hbm.at[pl.ds(sid * tper, tper)],
                             stage.at[pl.ds(sid * tper, tper)], sem_t).wait()
            plsc.subcore_barrier()
            base0 = wid * per_w

            def body(i, carry, idx_hbm=idx_hbm, out=out, base0=base0,
                     gw=gw, grp=grp):
                base = base0 + i * gw
                ic = []
                for j in range(grp):
                    ic.append(pltpu.async_copy(
                        idx_hbm.at[pl.ds(base + j * gch, gch)], idxs[j],
                        sem_i))
                for c in ic:
                    c.wait()
                gc = []
                for j in range(grp):
                    gc.append(pltpu.async_copy(stage.at[idxs[j]], rows[j],
                                               sem_g))
                for c in gc:
                    c.wait()
                oc = []
                for j in range(grp):
                    oc.append(pltpu.async_copy(
                        rows[j], out.at[pl.ds(base + j * gch, gch)], sem_i))
                for c in oc:
                    c.wait()
                return carry

            lax.fori_loop(0, n_grp, body, 0)
            plsc.subcore_barrier()

    return gather_k


SCH = 40
SGRP = 5
SST = 64


@functools.lru_cache(maxsize=None)
def _make_scatter_all(sets, sch=SCH):
    plans = []
    for R, segs in sets:
        seg_plans = []
        for E in segs:
            per_w = E // NW
            assert per_w % sch == 0
            n_ch = per_w // sch
            grp = max(g for g in range(1, SGRP + 1) if n_ch % g == 0)
            seg_plans.append((per_w, grp, n_ch // grp))
        P = R // NS
        S = min(P, SST)
        assert P % S == 0
        plans.append((R, tuple(seg_plans), P, S, P // S))
    r_max = max(R for R, _ in sets)
    n_args = sum(2 * len(segs) for _, segs in sets)
    mesh = plsc.VectorSubcoreMesh(**_SC_MESH)
    scratch = ([pltpu.VMEM((sch,), jnp.int32) for _ in range(SGRP)]
               + [pltpu.VMEM((sch, D), F32) for _ in range(SGRP)]
               + [pltpu.VMEM((SST, D), F32),
                  pltpu.VMEM_SHARED((r_max, D), F32),
                  pltpu.SemaphoreType.DMA, pltpu.SemaphoreType.DMA])
    out_type = tuple(jax.ShapeDtypeStruct((R, D), F32)
                     for R, _ in sets for _ in range(NC))

    @functools.partial(pl.kernel, out_type=out_type, mesh=mesh,
                       scratch_types=scratch)
    def scatter_k(*refs):
        args = refs[:n_args + 1]
        outs = refs[n_args + 1:n_args + 1 + 2 * len(sets)]
        scr = refs[n_args + 1 + 2 * len(sets):]
        idxs = scr[:SGRP]
        rows = scr[SGRP:2 * SGRP]
        stage, acc, sem_i, sem_s = scr[2 * SGRP:]
        zero_hbm = args[n_args]
        cid = lax.axis_index("c")
        sid = lax.axis_index("s")
        wid = sid * NC + cid

        ai = 0
        for si, (R, seg_plans, P, S, n_st) in enumerate(plans):
            out0, out1 = outs[2 * si], outs[2 * si + 1]
            pltpu.sync_copy(zero_hbm, stage)

            def zbody(k, carry, P=P, S=S):
                pltpu.sync_copy(stage.at[pl.ds(0, S)],
                                acc.at[pl.ds(sid * P + k * S, S)])
                return carry

            lax.fori_loop(0, n_st, zbody, 0)
            plsc.subcore_barrier()

            for per_w, grp, n_grp in seg_plans:
                rows_hbm, idx_hbm = args[ai], args[ai + 1]
                ai += 2
                gw = sch * grp
                base0 = wid * per_w

                def body(i, carry, rows_hbm=rows_hbm, idx_hbm=idx_hbm,
                         base0=base0, gw=gw, grp=grp):
                    base = base0 + i * gw
                    ic = []
                    for j in range(grp):
                        ic.append(pltpu.async_copy(
                            idx_hbm.at[pl.ds(base + j * sch, sch)], idxs[j],
                            sem_i))
                        ic.append(pltpu.async_copy(
                            rows_hbm.at[pl.ds(base + j * sch, sch)], rows[j],
                            sem_i))
                    for c in ic:
                        c.wait()
                    sc_ = []
                    for j in range(grp):
                        sc_.append(pltpu.async_copy(rows[j], acc.at[idxs[j]],
                                                    sem_s, add=True))
                    for c in sc_:
                        c.wait()
                    return carry

                lax.fori_loop(0, n_grp, body, 0)
            plsc.subcore_barrier()

            def obody(k, carry, out0=out0, out1=out1, P=P, S=S):
                st = sid * P + k * S
                pltpu.sync_copy(acc.at[pl.ds(st, S)], stage.at[pl.ds(0, S)])

                @pl.when(cid == 0)
                def _():
                    pltpu.sync_copy(stage.at[pl.ds(0, S)],
                                    out0.at[pl.ds(st, S)])

                @pl.when(cid == 1)
                def _():
                    pltpu.sync_copy(stage.at[pl.ds(0, S)],
                                    out1.at[pl.ds(st, S)])

                return carry

            lax.fori_loop(0, n_st, obody, 0)

    return scatter_k


def _ln(h, g, b):
    mu = jnp.mean(h, axis=-1, keepdims=True)
    d = h - mu
    var = jnp.mean(d * d, axis=-1, keepdims=True)
    return d * lax.rsqrt(var + EPS) * g + b


def _proj_body(x_ref, w_ref, b_ref, *out_refs):
    x = x_ref[...]
    for j, o_ref in enumerate(out_refs):
        o_ref[...] = jnp.dot(x, w_ref[j], preferred_element_type=F32) + b_ref[j]


def _proj(x, w_all, b_all):
    RB = 1000
    grid = (NNODE // RB,)
    return pl.pallas_call(
        _proj_body,
        grid=grid,
        in_specs=[
            pl.BlockSpec((RB, D), lambda i: (i, 0)),
            pl.BlockSpec((8, D, D), lambda i: (0, 0, 0)),
            pl.BlockSpec((8, 1, D), lambda i: (0, 0, 0)),
        ],
        out_specs=[pl.BlockSpec((RB, D), lambda i: (i, 0))] * 8,
        out_shape=[jax.ShapeDtypeStruct((NNODE, D), F32)] * 8,
    )(x, w_all, b_all)


def _edge_body(gs_ref, gr_ref, ef_ref, w1c_ref, w2_ref, b2_ref, g_ref, b_ref,
               new_ref, oute_ref):
    ef = ef_ref[...]
    h = gs_ref[...] + gr_ref[...] + jnp.dot(ef, w1c_ref[...],
                                            preferred_element_type=F32)
    h = jnp.maximum(h, 0.0)
    h2 = jnp.dot(h, w2_ref[...], preferred_element_type=F32) + b2_ref[...]
    new = _ln(h2, g_ref[...], b_ref[...])
    new_ref[...] = new
    oute_ref[...] = new + ef


def _edge_body_aliased(gs_ref, gr_ref, ef_ref, w1c_ref, w2_ref, b2_ref, g_ref,
                       b_ref, oute_prev_ref, new_ref, oute_ref):
    del oute_prev_ref
    _edge_body(gs_ref, gr_ref, ef_ref, w1c_ref, w2_ref, b2_ref, g_ref, b_ref,
               new_ref, oute_ref)


def _edge_mlp_mesh_half(gs, gr, ef, p, half, oute_prev):
    blk = 2000
    nb = (E_MESH // 4) // blk
    off = half * nb
    row = lambda i: (i, 0)
    eff = lambda i: (i + off, 0)
    full = lambda i: (0, 0)
    in_specs = [pl.BlockSpec((blk, D), row)] * 2 + [
        pl.BlockSpec((blk, D), eff),
        pl.BlockSpec((D, D), full), pl.BlockSpec((D, D), full),
        pl.BlockSpec((1, D), full), pl.BlockSpec((1, D), full),
        pl.BlockSpec((1, D), full)]
    args = [gs, gr, ef, p["W1"][2 * D:], p["W2"], p["b2"][None], p["g"][None],
            p["b"][None]]
    kwargs = {}
    body = _edge_body
    if half:
        in_specs.append(pl.BlockSpec(memory_space=pltpu.MemorySpace.HBM))
        args.append(oute_prev)
        kwargs["input_output_aliases"] = {8: 1}
        body = _edge_body_aliased
    return pl.pallas_call(
        body,
        grid=(nb,),
        in_specs=in_specs,
        out_specs=[pl.BlockSpec((blk, D), row), pl.BlockSpec((blk, D), eff)],
        out_shape=[jax.ShapeDtypeStruct((EQP, D), F32),
                   jax.ShapeDtypeStruct((E_MESH, D), F32)],
        **kwargs,
    )(*args)


def _edge_mlp(gs, gr, ef, p, E, blk):
    grid = (E // blk,)
    row = lambda i: (i, 0)
    full = lambda i: (0, 0)
    return pl.pallas_call(
        _edge_body,
        grid=grid,
        in_specs=[pl.BlockSpec((blk, D), row)] * 3 + [
            pl.BlockSpec((D, D), full),
            pl.BlockSpec((D, D), full),
            pl.BlockSpec((1, D), full),
            pl.BlockSpec((1, D), full),
            pl.BlockSpec((1, D), full),
        ],
        out_specs=[pl.BlockSpec((blk, D), row)] * 2,
        out_shape=[jax.ShapeDtypeStruct((E, D), F32)] * 2,
    )(gs, gr, ef, p["W1"][2 * D:], p["W2"], p["b2"][None], p["g"][None],
      p["b"][None])


def _mesh_node_body(x_ref, pm0_ref, pm1_ref, pm2_ref, pm3_ref,
                    pc0_ref, pc1_ref,
                    cA, cB, cC, cb1, cW2, cb2, cg, cbt,
                    dA, dB, db1, dW2, db2, dg, dbt, out_ref):
    x = x_ref[...]
    sm = (pm0_ref[...] + pm1_ref[...]) + (pm2_ref[...] + pm3_ref[...])
    sc = pc0_ref[...] + pc1_ref[...]
    h = jnp.dot(x, cA[...], preferred_element_type=F32)
    h += jnp.dot(sm, cB[...], preferred_element_type=F32)
    h += jnp.dot(sc, cC[...], preferred_element_type=F32)
    h = jnp.maximum(h + cb1[...], 0.0)
    u = _ln(jnp.dot(h, cW2[...], preferred_element_type=F32) + cb2[...],
            cg[...], cbt[...])
    h2 = jnp.dot(u, dA[...], preferred_element_type=F32)
    h2 += jnp.dot(sc, dB[...], preferred_element_type=F32)
    h2 = jnp.maximum(h2 + db1[...], 0.0)
    nd = _ln(jnp.dot(h2, dW2[...], preferred_element_type=F32) + db2[...],
             dg[...], dbt[...])
    out_ref[...] = nd + x


def _mesh_node(mesh_x, pm, pc, pcross, pdown):
    blk = 2000
    grid = (NMESH // blk,)
    row = lambda i: (i, 0)
    full = lambda i: (0, 0)
    wspec = pl.BlockSpec((D, D), full)
    vspec = pl.BlockSpec((1, D), full)
    return pl.pallas_call(
        _mesh_node_body,
        grid=grid,
        in_specs=[pl.BlockSpec((blk, D), row)] * 7 + [
            wspec, wspec, wspec, vspec, wspec, vspec, vspec, vspec,
            wspec, wspec, vspec, wspec, vspec, vspec, vspec,
        ],
        out_specs=pl.BlockSpec((blk, D), row),
        out_shape=jax.ShapeDtypeStruct((NMESH, D), F32),
    )(mesh_x, pm[0], pm[1], pm[2], pm[3], pc[0], pc[1],
      pcross["W1"][:D], pcross["W1"][D:2 * D], pcross["W1"][4 * D:],
      pcross["b1"][None], pcross["W2"], pcross["b2"][None],
      pcross["g"][None], pcross["b"][None],
      pdown["W1"][:D], pdown["W1"][D:], pdown["b1"][None], pdown["W2"],
      pdown["b2"][None], pdown["g"][None], pdown["b"][None])


def _hyper_node_body(x_ref, pi0_ref, pi1_ref, pn0_ref, pn1_ref,
                     uA, uB, ub1, uW2, ub2, ug, ubt,
                     vA, vB, vb1, vW2, vb2, vg, vbt, out_ref):
    x = x_ref[...]
    si = pi0_ref[...][:NHYPER] + pi1_ref[...][:NHYPER]
    sn = pn0_ref[...][:NHYPER] + pn1_ref[...][:NHYPER]
    h = jnp.dot(x, uA[...], preferred_element_type=F32)
    h += jnp.dot(si, uB[...], preferred_element_type=F32)
    h = jnp.maximum(h + ub1[...], 0.0)
    hu = _ln(jnp.dot(h, uW2[...], preferred_element_type=F32) + ub2[...],
             ug[...], ubt[...])
    h2 = jnp.dot(hu, vA[...], preferred_element_type=F32)
    h2 += jnp.dot(sn, vB[...], preferred_element_type=F32)
    h2 = jnp.maximum(h2 + vb1[...], 0.0)
    hc = _ln(jnp.dot(h2, vW2[...], preferred_element_type=F32) + vb2[...],
             vg[...], vbt[...])
    out_ref[...] = hc + x


def _hyper_node(hyper_x, pi, pn, pup, pcross, RH):
    wspec = pl.BlockSpec((D, D), lambda: (0, 0))
    vspec = pl.BlockSpec((1, D), lambda: (0, 0))
    pspec = pl.BlockSpec((RH, D), lambda: (0, 0))
    return pl.pallas_call(
        _hyper_node_body,
        grid=(),
        in_specs=[
            pl.BlockSpec((NHYPER, D), lambda: (0, 0)),
            pspec, pspec, pspec, pspec,
            wspec, wspec, vspec, wspec, vspec, vspec, vspec,
            wspec, wspec, vspec, wspec, vspec, vspec, vspec,
        ],
        out_specs=pl.BlockSpec((NHYPER, D), lambda: (0, 0)),
        out_shape=jax.ShapeDtypeStruct((NHYPER, D), F32),
    )(hyper_x, pi[0], pi[1], pn[0], pn[1],
      pup["W1"][:D], pup["W1"][D:], pup["b1"][None], pup["W2"],
      pup["b2"][None], pup["g"][None], pup["b"][None],
      pcross["W1"][:D], pcross["W1"][D:], pcross["b1"][None], pcross["W2"],
      pcross["b2"][None], pcross["g"][None], pcross["b"][None])


E_MESH = 320000
E_I2C = 10000
E_INTER = 16000
E_C2M = 10000
EQP = 81920
EP_I2C = 10240
EP_INTER = 20480
EP_C2M = 10240
R_MESH = 10240
R_HYP = 1024


def kernel(mesh_x, hyper_x, mesh_ef, i2c_ef, inter_ef, c2m_ef, params,
           mesh_s, mesh_r, i2c_s, i2c_r, inter_s, inter_r, c2m_s, c2m_r):
    p = params
    x = jnp.concatenate([mesh_x, hyper_x], axis=0)

    names = ("edge_mesh", "edge_i2c", "edge_inter", "edge_c2m")
    w_all = jnp.stack([w for nm in names
                       for w in (p[nm]["W1"][:D], p[nm]["W1"][D:2 * D])])
    b_all = jnp.stack([b for nm in names
                       for b in (p[nm]["b1"], jnp.zeros((D,), F32))])[:, None]
    tabs = _proj(x, w_all, b_all)

    def pad_idx(idx, ep, fill):
        return jnp.pad(idx, (0, ep - idx.shape[0]), constant_values=fill)

    def pad_rows(a, ep):
        return jnp.pad(a, ((0, ep - a.shape[0]), (0, 0)))

    i2c_sp = pad_idx(i2c_s, EP_I2C, 0)
    c2m_rp = pad_idx(c2m_r, EP_C2M, 0)
    i2c_efp = pad_rows(i2c_ef, EP_I2C)
    inter_efp = pad_rows(inter_ef, EP_INTER)
    c2m_efp = pad_rows(c2m_ef, EP_C2M)
    sidx_i2c = pad_idx(i2c_r - NMESH, EP_I2C, R_HYP - 1)
    sidx_int = pad_idx(inter_r - NMESH, EP_INTER, R_HYP - 1)
    gidx_int_s = pad_idx(inter_s - NMESH, EP_INTER, 0)
    gidx_c2m_s = pad_idx(c2m_s - NMESH, EP_C2M, 0)

    def hyp_tab(t):
        return jnp.pad(t[NMESH:], ((0, R_HYP - NHYPER), (0, 0)))

    eq = E_MESH // 4
    msq = [pad_idx(mesh_s[k * eq:(k + 1) * eq], EQP, 0) for k in range(4)]
    mrq = [pad_idx(mesh_r[k * eq:(k + 1) * eq], EQP, R_MESH - 1)
           for k in range(4)]
    jobs_small = ((EP_I2C, R_MESH), (EP_I2C, R_HYP), (EP_INTER, R_HYP),
                  (EP_INTER, R_HYP), (EP_C2M, R_HYP), (EP_C2M, R_MESH))
    gs_i, gr_i, gs_n, gr_n, gs_c, gr_c = _make_gather_all(jobs_small)(
        tabs[2], i2c_sp, hyp_tab(tabs[3]), sidx_i2c, hyp_tab(tabs[4]),
        gidx_int_s, hyp_tab(tabs[5]), sidx_int, hyp_tab(tabs[6]), gidx_c2m_s,
        tabs[7], c2m_rp)
    jobs_mesh = ((EQP, R_MESH), (EQP, R_MESH))
    gq = [_make_gather_all(jobs_mesh)(tabs[0], msq[k], tabs[1], mrq[k])
          for k in range(4)]

    new_i, oute_i = _edge_mlp(gs_i, gr_i, i2c_efp, p["edge_i2c"], EP_I2C, 2048)
    new_n, oute_n = _edge_mlp(gs_n, gr_n, inter_efp, p["edge_inter"],
                              EP_INTER, 2048)
    new_c, oute_c = _edge_mlp(gs_c, gr_c, c2m_efp, p["edge_c2m"], EP_C2M, 2048)
    new_q = []
    oute_m = None
    for k in range(4):
        nk, oute_m = _edge_mlp_mesh_half(gq[k][0], gq[k][1], mesh_ef,
                                         p["edge_mesh"], k, oute_m)
        new_q.append(nk)

    zeroL = jnp.zeros((SST, D), F32)
    sidx_c2m = pad_idx(c2m_r, EP_C2M, R_MESH - 1)
    pm = (_make_scatter_all(((R_MESH, (EQP, EQP, EQP)),))(
              new_q[0], mrq[0], new_q[1], mrq[1], new_q[2], mrq[2], zeroL)
          + _make_scatter_all(((R_MESH, (EQP,)),))(new_q[3], mrq[3], zeroL))
    parts = _make_scatter_all(((R_HYP, (EP_I2C,)), (R_HYP, (EP_INTER,)),
                               (R_MESH, (EP_C2M,))))(
        new_i, sidx_i2c, new_n, sidx_int, new_c, sidx_c2m, zeroL)
    pi, pn, pc = (parts[0:2], parts[2:4], parts[4:6])

    out_mesh = _mesh_node(mesh_x, pm, pc, p["node_cross"], p["node_down"])
    out_hyper = _hyper_node(hyper_x, pi, pn, p["hyper_up"], p["hyper_cross"],
                            R_HYP)

    return (out_mesh, out_hyper, oute_m, oute_i[:E_I2C], oute_n[:E_INTER],
            oute_c[:E_C2M])


# speedup vs baseline: 1.0845x; 1.0845x over previous
import functools

import jax
import jax.numpy as jnp
from jax import lax
from jax.experimental import pallas as pl
from jax.experimental.pallas import tpu as pltpu
from jax.experimental.pallas import tpu_sc as plsc

F32 = jnp.float32
BF16 = jnp.bfloat16
D = 128
NMESH = 10000
NHYPER = 1000
NNODE = NMESH + NHYPER
NC, NS = 2, 16
NW = NC * NS
CH = 128
EPS = 1e-5

_SC_MESH = dict(core_axis_name="c", subcore_axis_name="s",
                num_cores=NC, num_subcores=NS)


def _chunk_plan(per_worker):
    best = None
    for c in range(8, CH + 1, 8):
        if per_worker % c:
            continue
        n_ch = per_worker // c
        cap = min(8, 896 // (2 * c))
        gs = [g for g in range(1, cap + 1) if n_ch % g == 0]
        if not gs:
            continue
        g = max(gs)
        if best is None or (g, c) > (best[1], best[0]):
            best = (c, g, n_ch // g)
    return best


GCH = 40
GGRP = 5


@functools.lru_cache(maxsize=None)
def _make_gather_all(jobs, gch=GCH):
    plans = []
    for E, TR in jobs:
        per_w = E // NW
        assert per_w % gch == 0
        n_ch = per_w // gch
        grp = max(g for g in range(1, GGRP + 1) if n_ch % g == 0)
        assert TR % NS == 0
        plans.append((E, TR, per_w, grp, n_ch // grp, TR // NS))
    tr_max = max(TR for _, TR in jobs)
    mesh = plsc.VectorSubcoreMesh(**_SC_MESH)
    scratch = ([pltpu.VMEM((gch,), jnp.int32) for _ in range(GGRP)]
               + [pltpu.VMEM((gch, D), F32) for _ in range(GGRP)]
               + [pltpu.VMEM_SHARED((tr_max, D), F32),
                  pltpu.SemaphoreType.DMA, pltpu.SemaphoreType.DMA,
                  pltpu.SemaphoreType.DMA])
    out_type = tuple(jax.ShapeDtypeStruct((E, D), F32) for E, _ in jobs)

    @functools.partial(pl.kernel, out_type=out_type, mesh=mesh,
                       scratch_types=scratch)
    def gather_k(*refs):
        nj = len(jobs)
        args = refs[:2 * nj]
        outs = refs[2 * nj:3 * nj]
        scr = refs[3 * nj:]
        idxs = scr[:GGRP]
        rows = scr[GGRP:2 * GGRP]
        stage, sem_t, sem_i, sem_g = scr[2 * GGRP:]
        cid = lax.axis_index("c")
        sid = lax.axis_index("s")
        wid = sid * NC + cid

        for ji, (E, TR, per_w, grp, n_grp, tper) in enumerate(plans):
            tab_hbm, idx_hbm = args[2 * ji], args[2 * ji + 1]
            out = outs[ji]
            gw = gch * grp
            pltpu.async_copy(tab_hbm.at[pl.ds(sid * tper, tper)],
                             stage.at[pl.ds(sid * tper, tper)], sem_t).wait()
            plsc.subcore_barrier()
            base0 = wid * per_w

            def body(i, carry, idx_hbm=idx_hbm, out=out, base0=base0,
                     gw=gw, grp=grp):
                base = base0 + i * gw
                ic = []
                for j in range(grp):
                    ic.append(pltpu.async_copy(
                        idx_hbm.at[pl.ds(base + j * gch, gch)], idxs[j],
                        sem_i))
                for c in ic:
                    c.wait()
                gc = []
                for j in range(grp):
                    gc.append(pltpu.async_copy(stage.at[idxs[j]], rows[j],
                                               sem_g))
                for c in gc:
                    c.wait()
                oc = []
                for j in range(grp):
                    oc.append(pltpu.async_copy(
                        rows[j], out.at[pl.ds(base + j * gch, gch)], sem_i))
                for c in oc:
                    c.wait()
                return carry

            lax.fori_loop(0, n_grp, body, 0)
            plsc.subcore_barrier()

    return gather_k


SCH = 40
SGRP = 5
SST = 64


@functools.lru_cache(maxsize=None)
def _make_scatter_all(sets, sch=SCH):
    plans = []
    for R, segs in sets:
        seg_plans = []
        for E in segs:
            per_w = E // NW
            assert per_w % sch == 0
            n_ch = per_w // sch
            grp = max(g for g in range(1, SGRP + 1) if n_ch % g == 0)
            seg_plans.append((per_w, grp, n_ch // grp))
        P = R // NS
        S = min(P, SST)
        assert P % S == 0
        plans.append((R, tuple(seg_plans), P, S, P // S))
    r_max = max(R for R, _ in sets)
    n_args = sum(2 * len(segs) for _, segs in sets)
    mesh = plsc.VectorSubcoreMesh(**_SC_MESH)
    scratch = ([pltpu.VMEM((sch,), jnp.int32) for _ in range(SGRP)]
               + [pltpu.VMEM((sch, D), F32) for _ in range(SGRP)]
               + [pltpu.VMEM((SST, D), F32),
                  pltpu.VMEM_SHARED((r_max, D), F32),
                  pltpu.SemaphoreType.DMA, pltpu.SemaphoreType.DMA])
    out_type = tuple(jax.ShapeDtypeStruct((R, D), F32)
                     for R, _ in sets for _ in range(NC))

    @functools.partial(pl.kernel, out_type=out_type, mesh=mesh,
                       scratch_types=scratch)
    def scatter_k(*refs):
        args = refs[:n_args + 1]
        outs = refs[n_args + 1:n_args + 1 + 2 * len(sets)]
        scr = refs[n_args + 1 + 2 * len(sets):]
        idxs = scr[:SGRP]
        rows = scr[SGRP:2 * SGRP]
        stage, acc, sem_i, sem_s = scr[2 * SGRP:]
        zero_hbm = args[n_args]
        cid = lax.axis_index("c")
        sid = lax.axis_index("s")
        wid = sid * NC + cid

        ai = 0
        for si, (R, seg_plans, P, S, n_st) in enumerate(plans):
            out0, out1 = outs[2 * si], outs[2 * si + 1]
            pltpu.sync_copy(zero_hbm, stage)

            def zbody(k, carry, P=P, S=S):
                pltpu.sync_copy(stage.at[pl.ds(0, S)],
                                acc.at[pl.ds(sid * P + k * S, S)])
                return carry

            lax.fori_loop(0, n_st, zbody, 0)
            plsc.subcore_barrier()

            for per_w, grp, n_grp in seg_plans:
                rows_hbm, idx_hbm = args[ai], args[ai + 1]
                ai += 2
                gw = sch * grp
                base0 = wid * per_w

                def body(i, carry, rows_hbm=rows_hbm, idx_hbm=idx_hbm,
                         base0=base0, gw=gw, grp=grp):
                    base = base0 + i * gw
                    ic = []
                    for j in range(grp):
                        ic.append(pltpu.async_copy(
                            idx_hbm.at[pl.ds(base + j * sch, sch)], idxs[j],
                            sem_i))
                        ic.append(pltpu.async_copy(
                            rows_hbm.at[pl.ds(base + j * sch, sch)], rows[j],
                            sem_i))
                    for c in ic:
                        c.wait()
                    sc_ = []
                    for j in range(grp):
                        sc_.append(pltpu.async_copy(rows[j], acc.at[idxs[j]],
                                                    sem_s, add=True))
                    for c in sc_:
                        c.wait()
                    return carry

                lax.fori_loop(0, n_grp, body, 0)
            plsc.subcore_barrier()

            def obody(k, carry, out0=out0, out1=out1, P=P, S=S):
                st = sid * P + k * S
                pltpu.sync_copy(acc.at[pl.ds(st, S)], stage.at[pl.ds(0, S)])

                @pl.when(cid == 0)
                def _():
                    pltpu.sync_copy(stage.at[pl.ds(0, S)],
                                    out0.at[pl.ds(st, S)])

                @pl.when(cid == 1)
                def _():
                    pltpu.sync_copy(stage.at[pl.ds(0, S)],
                                    out1.at[pl.ds(st, S)])

                return carry

            lax.fori_loop(0, n_st, obody, 0)

    return scatter_k


def _ln(h, g, b):
    mu = jnp.mean(h, axis=-1, keepdims=True)
    d = h - mu
    var = jnp.mean(d * d, axis=-1, keepdims=True)
    return d * lax.rsqrt(var + EPS) * g + b


def _proj_body(x_ref, w_ref, b_ref, *out_refs):
    x = x_ref[...]
    for j, o_ref in enumerate(out_refs):
        o_ref[...] = jnp.dot(x, w_ref[j], preferred_element_type=F32) + b_ref[j]


def _proj(x, w_all, b_all):
    RB = 1000
    grid = (NNODE // RB,)
    return pl.pallas_call(
        _proj_body,
        grid=grid,
        in_specs=[
            pl.BlockSpec((RB, D), lambda i: (i, 0)),
            pl.BlockSpec((8, D, D), lambda i: (0, 0, 0)),
            pl.BlockSpec((8, 1, D), lambda i: (0, 0, 0)),
        ],
        out_specs=[pl.BlockSpec((RB, D), lambda i: (i, 0))] * 8,
        out_shape=[jax.ShapeDtypeStruct((NNODE, D), F32)] * 8,
    )(x, w_all, b_all)


def _edge_body(gs_ref, gr_ref, ef_ref, w1c_ref, w2_ref, b2_ref, g_ref, b_ref,
               new_ref, oute_ref):
    ef = ef_ref[...]
    h = gs_ref[...] + gr_ref[...] + jnp.dot(ef, w1c_ref[...],
                                            preferred_element_type=F32)
    h = jnp.maximum(h, 0.0)
    h2 = jnp.dot(h, w2_ref[...], preferred_element_type=F32) + b2_ref[...]
    new = _ln(h2, g_ref[...], b_ref[...])
    new_ref[...] = new
    oute_ref[...] = new + ef


def _edge_body_aliased(gs_ref, gr_ref, ef_ref, w1c_ref, w2_ref, b2_ref, g_ref,
                       b_ref, oute_prev_ref, new_ref, oute_ref):
    del oute_prev_ref
    _edge_body(gs_ref, gr_ref, ef_ref, w1c_ref, w2_ref, b2_ref, g_ref, b_ref,
               new_ref, oute_ref)


def _edge_mlp_mesh_half(gs, gr, ef, p, half, oute_prev):
    blk = 2000
    nb = (E_MESH // 2) // blk
    off = half * nb
    row = lambda i: (i, 0)
    eff = lambda i: (i + off, 0)
    full = lambda i: (0, 0)
    in_specs = [pl.BlockSpec((blk, D), row)] * 2 + [
        pl.BlockSpec((blk, D), eff),
        pl.BlockSpec((D, D), full), pl.BlockSpec((D, D), full),
        pl.BlockSpec((1, D), full), pl.BlockSpec((1, D), full),
        pl.BlockSpec((1, D), full)]
    args = [gs, gr, ef, p["W1"][2 * D:], p["W2"], p["b2"][None], p["g"][None],
            p["b"][None]]
    kwargs = {}
    body = _edge_body
    if half:
        in_specs.append(pl.BlockSpec(memory_space=pltpu.MemorySpace.HBM))
        args.append(oute_prev)
        kwargs["input_output_aliases"] = {8: 1}
        body = _edge_body_aliased
    return pl.pallas_call(
        body,
        grid=(nb,),
        in_specs=in_specs,
        out_specs=[pl.BlockSpec((blk, D), row), pl.BlockSpec((blk, D), eff)],
        out_shape=[jax.ShapeDtypeStruct((E_MESH // 2, D), F32),
                   jax.ShapeDtypeStruct((E_MESH, D), F32)],
        **kwargs,
    )(*args)


def _edge_mlp(gs, gr, ef, p, E, blk):
    grid = (E // blk,)
    row = lambda i: (i, 0)
    full = lambda i: (0, 0)
    return pl.pallas_call(
        _edge_body,
        grid=grid,
        in_specs=[pl.BlockSpec((blk, D), row)] * 3 + [
            pl.BlockSpec((D, D), full),
            pl.BlockSpec((D, D), full),
            pl.BlockSpec((1, D), full),
            pl.BlockSpec((1, D), full),
            pl.BlockSpec((1, D), full),
        ],
        out_specs=[pl.BlockSpec((blk, D), row)] * 2,
        out_shape=[jax.ShapeDtypeStruct((E, D), F32)] * 2,
    )(gs, gr, ef, p["W1"][2 * D:], p["W2"], p["b2"][None], p["g"][None],
      p["b"][None])


def _mesh_node_body(x_ref, pm0_ref, pm1_ref, pm2_ref, pm3_ref,
                    pc0_ref, pc1_ref,
                    cA, cB, cC, cb1, cW2, cb2, cg, cbt,
                    dA, dB, db1, dW2, db2, dg, dbt, out_ref):
    x = x_ref[...]
    sm = (pm0_ref[...] + pm1_ref[...]) + (pm2_ref[...] + pm3_ref[...])
    sc = pc0_ref[...] + pc1_ref[...]
    h = jnp.dot(x, cA[...], preferred_element_type=F32)
    h += jnp.dot(sm, cB[...], preferred_element_type=F32)
    h += jnp.dot(sc, cC[...], preferred_element_type=F32)
    h = jnp.maximum(h + cb1[...], 0.0)
    u = _ln(jnp.dot(h, cW2[...], preferred_element_type=F32) + cb2[...],
            cg[...], cbt[...])
    h2 = jnp.dot(u, dA[...], preferred_element_type=F32)
    h2 += jnp.dot(sc, dB[...], preferred_element_type=F32)
    h2 = jnp.maximum(h2 + db1[...], 0.0)
    nd = _ln(jnp.dot(h2, dW2[...], preferred_element_type=F32) + db2[...],
             dg[...], dbt[...])
    out_ref[...] = nd + x


def _mesh_node(mesh_x, pm, pc, pcross, pdown):
    blk = 2000
    grid = (NMESH // blk,)
    row = lambda i: (i, 0)
    full = lambda i: (0, 0)
    wspec = pl.BlockSpec((D, D), full)
    vspec = pl.BlockSpec((1, D), full)
    return pl.pallas_call(
        _mesh_node_body,
        grid=grid,
        in_specs=[pl.BlockSpec((blk, D), row)] * 7 + [
            wspec, wspec, wspec, vspec, wspec, vspec, vspec, vspec,
            wspec, wspec, vspec, wspec, vspec, vspec, vspec,
        ],
        out_specs=pl.BlockSpec((blk, D), row),
        out_shape=jax.ShapeDtypeStruct((NMESH, D), F32),
    )(mesh_x, pm[0], pm[1], pm[2], pm[3], pc[0], pc[1],
      pcross["W1"][:D], pcross["W1"][D:2 * D], pcross["W1"][4 * D:],
      pcross["b1"][None], pcross["W2"], pcross["b2"][None],
      pcross["g"][None], pcross["b"][None],
      pdown["W1"][:D], pdown["W1"][D:], pdown["b1"][None], pdown["W2"],
      pdown["b2"][None], pdown["g"][None], pdown["b"][None])


def _hyper_node_body(x_ref, pi0_ref, pi1_ref, pn0_ref, pn1_ref,
                     uA, uB, ub1, uW2, ub2, ug, ubt,
                     vA, vB, vb1, vW2, vb2, vg, vbt, out_ref):
    x = x_ref[...]
    si = pi0_ref[...][:NHYPER] + pi1_ref[...][:NHYPER]
    sn = pn0_ref[...][:NHYPER] + pn1_ref[...][:NHYPER]
    h = jnp.dot(x, uA[...], preferred_element_type=F32)
    h += jnp.dot(si, uB[...], preferred_element_type=F32)
    h = jnp.maximum(h + ub1[...], 0.0)
    hu = _ln(jnp.dot(h, uW2[...], preferred_element_type=F32) + ub2[...],
             ug[...], ubt[...])
    h2 = jnp.dot(hu, vA[...], preferred_element_type=F32)
    h2 += jnp.dot(sn, vB[...], preferred_element_type=F32)
    h2 = jnp.maximum(h2 + vb1[...], 0.0)
    hc = _ln(jnp.dot(h2, vW2[...], preferred_element_type=F32) + vb2[...],
             vg[...], vbt[...])
    out_ref[...] = hc + x


def _hyper_node(hyper_x, pi, pn, pup, pcross, RH):
    wspec = pl.BlockSpec((D, D), lambda: (0, 0))
    vspec = pl.BlockSpec((1, D), lambda: (0, 0))
    pspec = pl.BlockSpec((RH, D), lambda: (0, 0))
    return pl.pallas_call(
        _hyper_node_body,
        grid=(),
        in_specs=[
            pl.BlockSpec((NHYPER, D), lambda: (0, 0)),
            pspec, pspec, pspec, pspec,
            wspec, wspec, vspec, wspec, vspec, vspec, vspec,
            wspec, wspec, vspec, wspec, vspec, vspec, vspec,
        ],
        out_specs=pl.BlockSpec((NHYPER, D), lambda: (0, 0)),
        out_shape=jax.ShapeDtypeStruct((NHYPER, D), F32),
    )(hyper_x, pi[0], pi[1], pn[0], pn[1],
      pup["W1"][:D], pup["W1"][D:], pup["b1"][None], pup["W2"],
      pup["b2"][None], pup["g"][None], pup["b"][None],
      pcross["W1"][:D], pcross["W1"][D:], pcross["b1"][None], pcross["W2"],
      pcross["b2"][None], pcross["g"][None], pcross["b"][None])


E_MESH = 320000
E_I2C = 10000
E_INTER = 16000
E_C2M = 10000
EQP = 81920
EP_I2C = 10240
EP_INTER = 20480
EP_C2M = 10240
R_MESH = 10240
R_HYP = 1024


def kernel(mesh_x, hyper_x, mesh_ef, i2c_ef, inter_ef, c2m_ef, params,
           mesh_s, mesh_r, i2c_s, i2c_r, inter_s, inter_r, c2m_s, c2m_r):
    p = params
    x = jnp.concatenate([mesh_x, hyper_x], axis=0)

    names = ("edge_mesh", "edge_i2c", "edge_inter", "edge_c2m")
    w_all = jnp.stack([w for nm in names
                       for w in (p[nm]["W1"][:D], p[nm]["W1"][D:2 * D])])
    b_all = jnp.stack([b for nm in names
                       for b in (p[nm]["b1"], jnp.zeros((D,), F32))])[:, None]
    tabs = _proj(x, w_all, b_all)

    def pad_idx(idx, ep, fill):
        return jnp.pad(idx, (0, ep - idx.shape[0]), constant_values=fill)

    def pad_rows(a, ep):
        return jnp.pad(a, ((0, ep - a.shape[0]), (0, 0)))

    i2c_sp = pad_idx(i2c_s, EP_I2C, 0)
    c2m_rp = pad_idx(c2m_r, EP_C2M, 0)
    i2c_efp = pad_rows(i2c_ef, EP_I2C)
    inter_efp = pad_rows(inter_ef, EP_INTER)
    c2m_efp = pad_rows(c2m_ef, EP_C2M)
    sidx_i2c = pad_idx(i2c_r - NMESH, EP_I2C, R_HYP - 1)
    sidx_int = pad_idx(inter_r - NMESH, EP_INTER, R_HYP - 1)
    gidx_int_s = pad_idx(inter_s - NMESH, EP_INTER, 0)
    gidx_c2m_s = pad_idx(c2m_s - NMESH, EP_C2M, 0)

    def hyp_tab(t):
        return jnp.pad(t[NMESH:], ((0, R_HYP - NHYPER), (0, 0)))

    eh = E_MESH // 2
    ms1, ms2 = mesh_s[:eh], mesh_s[eh:]
    mr1, mr2 = mesh_r[:eh], mesh_r[eh:]
    jobs_small = ((EP_I2C, R_MESH), (EP_I2C, R_HYP), (EP_INTER, R_HYP),
                  (EP_INTER, R_HYP), (EP_C2M, R_HYP), (EP_C2M, R_MESH))
    gs_i, gr_i, gs_n, gr_n, gs_c, gr_c = _make_gather_all(jobs_small)(
        tabs[2], i2c_sp, hyp_tab(tabs[3]), sidx_i2c, hyp_tab(tabs[4]),
        gidx_int_s, hyp_tab(tabs[5]), sidx_int, hyp_tab(tabs[6]), gidx_c2m_s,
        tabs[7], c2m_rp)
    jobs_mesh = ((eh, R_MESH), (eh, R_MESH))
    gs_m1, gr_m1 = _make_gather_all(jobs_mesh)(tabs[0], ms1, tabs[1], mr1)
    gs_m2, gr_m2 = _make_gather_all(jobs_mesh)(tabs[0], ms2, tabs[1], mr2)

    new_i, oute_i = _edge_mlp(gs_i, gr_i, i2c_efp, p["edge_i2c"], EP_I2C, 2048)
    new_n, oute_n = _edge_mlp(gs_n, gr_n, inter_efp, p["edge_inter"],
                              EP_INTER, 2048)
    new_c, oute_c = _edge_mlp(gs_c, gr_c, c2m_efp, p["edge_c2m"], EP_C2M, 2048)
    new_m1, oute_h = _edge_mlp_mesh_half(gs_m1, gr_m1, mesh_ef,
                                         p["edge_mesh"], 0, None)
    new_m2, oute_m = _edge_mlp_mesh_half(gs_m2, gr_m2, mesh_ef,
                                         p["edge_mesh"], 1, oute_h)

    zeroL = jnp.zeros((SST, D), F32)
    sidx_c2m = pad_idx(c2m_r, EP_C2M, R_MESH - 1)
    scat_mesh = _make_scatter_all(((R_MESH, (eh,)),))
    pm = (scat_mesh(new_m1, mr1, zeroL) + scat_mesh(new_m2, mr2, zeroL))
    parts = _make_scatter_all(((R_HYP, (EP_I2C,)), (R_HYP, (EP_INTER,)),
                               (R_MESH, (EP_C2M,))))(
        new_i, sidx_i2c, new_n, sidx_int, new_c, sidx_c2m, zeroL)
    pi, pn, pc = (parts[0:2], parts[2:4], parts[4:6])

    out_mesh = _mesh_node(mesh_x, pm, pc, p["node_cross"], p["node_down"])
    out_hyper = _hyper_node(hyper_x, pi, pn, p["hyper_up"], p["hyper_cross"],
                            R_HYP)

    return (out_mesh, out_hyper, oute_m, oute_i[:E_I2C], oute_n[:E_INTER],
            oute_c[:E_C2M])
